# baseline (device time: 39929 ns/iter reference)
import jax
import jax.numpy as jnp
from jax import lax
from jax.experimental import pallas as pl
from jax.experimental.pallas import tpu as pltpu

T = 512
D = 512
V_PER = 4096
N_Y = 4


def kernel(ids, E):
    ids2 = ids.reshape(T, 1)

    def body(ids_ref, e_ref, out_ref, sbuf, rbuf, ssem, rsem):
        my_x = lax.axis_index("x")
        my_y = lax.axis_index("y")
        my_z = lax.axis_index("z")

        lid = ids_ref[:, :] - my_y * V_PER
        iota = lax.broadcasted_iota(jnp.int32, (T, V_PER), 1)
        onehot = (iota == lid).astype(jnp.bfloat16)
        eb = e_ref[:, :].astype(jnp.bfloat16)
        out_ref[:, :] = jnp.dot(
            onehot, eb, preferred_element_type=jnp.float32
        )

        for s, peer_off in enumerate((1, 2)):
            peer_y = jnp.bitwise_xor(my_y, peer_off)
            sbuf[s] = out_ref[:, :].astype(jnp.bfloat16)
            rdma = pltpu.make_async_remote_copy(
                src_ref=sbuf.at[s],
                dst_ref=rbuf.at[s],
                send_sem=ssem.at[s],
                recv_sem=rsem.at[s],
                device_id=(my_x, peer_y, my_z),
                device_id_type=pl.DeviceIdType.MESH,
            )
            rdma.start()
            rdma.wait()
            out_ref[:, :] = out_ref[:, :] + rbuf[s].astype(jnp.float32)

    return pl.pallas_call(
        body,
        out_shape=jax.ShapeDtypeStruct((T, D), jnp.float32),
        in_specs=[
            pl.BlockSpec(memory_space=pltpu.VMEM),
            pl.BlockSpec(memory_space=pltpu.VMEM),
        ],
        out_specs=pl.BlockSpec(memory_space=pltpu.VMEM),
        scratch_shapes=[
            pltpu.VMEM((2, T, D), jnp.bfloat16),
            pltpu.VMEM((2, T, D), jnp.bfloat16),
            pltpu.SemaphoreType.DMA((2,)),
            pltpu.SemaphoreType.DMA((2,)),
        ],
    )(ids2, E)


# device time: 31563 ns/iter; 1.2651x vs baseline; 1.2651x over previous
import jax
import jax.numpy as jnp
from jax import lax
from jax.experimental import pallas as pl
from jax.experimental.pallas import tpu as pltpu

T = 512
T2 = T // 2
D = 512
V_PER = 4096
N_Y = 4


def kernel(ids, E):
    ids2 = ids.reshape(T, 1)

    def body(ids_ref, e_ref, out_ref, pbuf, rl, rr, r2l, r2r, ssem, rsem):
        my_x = lax.axis_index("x")
        my_y = lax.axis_index("y")
        my_z = lax.axis_index("z")
        right = (my_x, (my_y + 1) % N_Y, my_z)
        left = (my_x, (my_y + 3) % N_Y, my_z)

        barrier_sem = pltpu.get_barrier_semaphore()
        for nbr in (left, right):
            pl.semaphore_signal(
                barrier_sem, inc=1,
                device_id=nbr, device_id_type=pl.DeviceIdType.MESH,
            )
        pl.semaphore_wait(barrier_sem, 2)

        lid = ids_ref[:, :] - my_y * V_PER
        iota = lax.broadcasted_iota(jnp.int32, (T, V_PER), 1)
        onehot = (iota == lid).astype(jnp.bfloat16)
        eb = e_ref[:, :].astype(jnp.bfloat16)
        p = jnp.dot(onehot, eb, preferred_element_type=jnp.float32)
        out_ref[:, :] = p
        pbuf[:, :] = p.astype(jnp.bfloat16)

        cw1 = pltpu.make_async_remote_copy(
            src_ref=pbuf, dst_ref=rl,
            send_sem=ssem.at[0], recv_sem=rsem.at[0],
            device_id=right, device_id_type=pl.DeviceIdType.MESH,
        )
        ccw1 = pltpu.make_async_remote_copy(
            src_ref=pbuf, dst_ref=rr,
            send_sem=ssem.at[1], recv_sem=rsem.at[1],
            device_id=left, device_id_type=pl.DeviceIdType.MESH,
        )
        cw1.start()
        ccw1.start()
        cw1.wait()
        ccw1.wait()

        cw2 = pltpu.make_async_remote_copy(
            src_ref=rl.at[pl.ds(0, T2)], dst_ref=r2l,
            send_sem=ssem.at[2], recv_sem=rsem.at[2],
            device_id=right, device_id_type=pl.DeviceIdType.MESH,
        )
        ccw2 = pltpu.make_async_remote_copy(
            src_ref=rr.at[pl.ds(T2, T2)], dst_ref=r2r,
            send_sem=ssem.at[3], recv_sem=rsem.at[3],
            device_id=left, device_id_type=pl.DeviceIdType.MESH,
        )
        cw2.start()
        ccw2.start()

        out_ref[:, :] = (
            out_ref[:, :]
            + rl[:, :].astype(jnp.float32)
            + rr[:, :].astype(jnp.float32)
        )

        cw2.wait()
        ccw2.wait()
        out_ref[pl.ds(0, T2), :] = (
            out_ref[pl.ds(0, T2), :] + r2l[:, :].astype(jnp.float32)
        )
        out_ref[pl.ds(T2, T2), :] = (
            out_ref[pl.ds(T2, T2), :] + r2r[:, :].astype(jnp.float32)
        )

    return pl.pallas_call(
        body,
        out_shape=jax.ShapeDtypeStruct((T, D), jnp.float32),
        in_specs=[
            pl.BlockSpec(memory_space=pltpu.VMEM),
            pl.BlockSpec(memory_space=pltpu.VMEM),
        ],
        out_specs=pl.BlockSpec(memory_space=pltpu.VMEM),
        scratch_shapes=[
            pltpu.VMEM((T, D), jnp.bfloat16),
            pltpu.VMEM((T, D), jnp.bfloat16),
            pltpu.VMEM((T, D), jnp.bfloat16),
            pltpu.VMEM((T2, D), jnp.bfloat16),
            pltpu.VMEM((T2, D), jnp.bfloat16),
            pltpu.SemaphoreType.DMA((4,)),
            pltpu.SemaphoreType.DMA((4,)),
        ],
        compiler_params=pltpu.CompilerParams(collective_id=0),
    )(ids2, E)


# device time: 24274 ns/iter; 1.6449x vs baseline; 1.3003x over previous
import os

import jax
import jax.numpy as jnp
from jax import lax
from jax.experimental import pallas as pl
from jax.experimental.pallas import tpu as pltpu

_VARIANT = os.environ.get("SCB_VARIANT", "full")

T = 512
D = 512
V_PER = 4096
N_Y = 4
N_Z = 4
R = T // 8


def kernel(ids, E):
    ids2 = ids.reshape(T, 1)

    def body(ids_ref, e_ref, out_ref, yb, zb, xb, ys, yr, zs, zr, xs, xr):
        my_x = lax.axis_index("x")
        my_y = lax.axis_index("y")
        my_z = lax.axis_index("z")
        s = my_x * N_Z + my_z
        row0 = s * R

        if _VARIANT != "compute":
            barrier_sem = pltpu.get_barrier_semaphore()
            for k in range(1, N_Y):
                pl.semaphore_signal(
                    barrier_sem, inc=1,
                    device_id=(my_x, (my_y + k) % N_Y, my_z),
                    device_id_type=pl.DeviceIdType.MESH,
                )
                pl.semaphore_signal(
                    barrier_sem, inc=1,
                    device_id=(my_x, my_y, (my_z + k) % N_Z),
                    device_id_type=pl.DeviceIdType.MESH,
                )
            pl.semaphore_signal(
                barrier_sem, inc=1,
                device_id=(1 - my_x, my_y, my_z),
                device_id_type=pl.DeviceIdType.MESH,
            )
            pl.semaphore_wait(barrier_sem, 7)

        if _VARIANT != "comm":
            lid = ids_ref[pl.ds(row0, R), :] - my_y * V_PER
            iota = lax.broadcasted_iota(jnp.int32, (R, V_PER), 1)
            onehot = (iota == lid).astype(jnp.bfloat16)
            eb = e_ref[:, :].astype(jnp.bfloat16)
            p = jnp.dot(onehot, eb, preferred_element_type=jnp.float32)
            yb[my_y] = p.astype(jnp.bfloat16)
        else:
            yb[my_y] = jnp.zeros((R, D), jnp.bfloat16)
        if _VARIANT == "compute":
            out_ref[:, :] = jnp.zeros((T, D), jnp.float32)
            out_ref[pl.ds(row0, R), :] = yb[my_y].astype(jnp.float32)
            return

        y_sends = []
        for k in range(1, N_Y):
            r = pltpu.make_async_remote_copy(
                src_ref=yb.at[my_y], dst_ref=yb.at[my_y],
                send_sem=ys.at[k - 1], recv_sem=yr.at[my_y],
                device_id=(my_x, (my_y + k) % N_Y, my_z),
                device_id_type=pl.DeviceIdType.MESH,
            )
            r.start()
            y_sends.append(r)
        for k in range(1, N_Y):
            src_y = (my_y + k) % N_Y
            pltpu.make_async_remote_copy(
                src_ref=yb.at[my_y], dst_ref=yb.at[src_y],
                send_sem=ys.at[k - 1], recv_sem=yr.at[src_y],
                device_id=(my_x, src_y, my_z),
                device_id_type=pl.DeviceIdType.MESH,
            ).wait_recv()
        fin = (
            yb[0].astype(jnp.float32) + yb[1].astype(jnp.float32)
            + yb[2].astype(jnp.float32) + yb[3].astype(jnp.float32)
        )
        zb[my_z] = fin.astype(jnp.bfloat16)

        z_sends = []
        for k in range(1, N_Z):
            r = pltpu.make_async_remote_copy(
                src_ref=zb.at[my_z], dst_ref=zb.at[my_z],
                send_sem=zs.at[k - 1], recv_sem=zr.at[my_z],
                device_id=(my_x, my_y, (my_z + k) % N_Z),
                device_id_type=pl.DeviceIdType.MESH,
            )
            r.start()
            z_sends.append(r)
        out_ref[pl.ds(row0, R), :] = fin
        for k in range(1, N_Z):
            src_z = (my_z + k) % N_Z
            pltpu.make_async_remote_copy(
                src_ref=zb.at[my_z], dst_ref=zb.at[src_z],
                send_sem=zs.at[k - 1], recv_sem=zr.at[src_z],
                device_id=(my_x, my_y, src_z),
                device_id_type=pl.DeviceIdType.MESH,
            ).wait_recv()

        xrdma = pltpu.make_async_remote_copy(
            src_ref=zb, dst_ref=xb,
            send_sem=xs.at[0], recv_sem=xr.at[0],
            device_id=(1 - my_x, my_y, my_z),
            device_id_type=pl.DeviceIdType.MESH,
        )
        xrdma.start()
        for j in range(N_Z):
            out_ref[pl.ds((my_x * N_Z + j) * R, R), :] = (
                zb[j].astype(jnp.float32)
            )
        xrdma.wait()
        for j in range(N_Z):
            out_ref[pl.ds(((1 - my_x) * N_Z + j) * R, R), :] = (
                xb[j].astype(jnp.float32)
            )
        for r in y_sends:
            r.wait_send()
        for r in z_sends:
            r.wait_send()

    return pl.pallas_call(
        body,
        out_shape=jax.ShapeDtypeStruct((T, D), jnp.float32),
        in_specs=[
            pl.BlockSpec(memory_space=pltpu.VMEM),
            pl.BlockSpec(memory_space=pltpu.VMEM),
        ],
        out_specs=pl.BlockSpec(memory_space=pltpu.VMEM),
        scratch_shapes=[
            pltpu.VMEM((N_Y, R, D), jnp.bfloat16),
            pltpu.VMEM((N_Z, R, D), jnp.bfloat16),
            pltpu.VMEM((N_Z, R, D), jnp.bfloat16),
            pltpu.SemaphoreType.DMA((N_Y - 1,)),
            pltpu.SemaphoreType.DMA((N_Y,)),
            pltpu.SemaphoreType.DMA((N_Z - 1,)),
            pltpu.SemaphoreType.DMA((N_Z,)),
            pltpu.SemaphoreType.DMA((1,)),
            pltpu.SemaphoreType.DMA((1,)),
        ],
        compiler_params=(
            pltpu.CompilerParams(collective_id=0)
            if _VARIANT != "compute"
            else pltpu.CompilerParams()
        ),
    )(ids2, E)


# device time: 11343 ns/iter; 3.5201x vs baseline; 2.1400x over previous
import os

import jax
import jax.numpy as jnp
from jax import lax
from jax.experimental import pallas as pl
from jax.experimental.pallas import tpu as pltpu

_VARIANT = os.environ.get("SCB_VARIANT", "full")

T = 512
D = 512
V_PER = 4096
N_X = 2
N_Y = 4
N_Z = 4
N_S = N_X * N_Z
R = T // N_S
N_GSEM = 8


def kernel(ids, E):
    ids2 = ids.reshape(T, 1)

    def body(idv_ref, ids_ref, e_ref, out_ref, g, yb, fb, gsem, ys, yr, fs, fr):
        my_x = lax.axis_index("x")
        my_y = lax.axis_index("y")
        my_z = lax.axis_index("z")
        my_s = my_x * N_Z + my_z
        row0 = my_s * R

        def xz_peers():
            peers = [(1 - my_x, my_y, my_z)]
            for k in range(1, N_Z):
                peers.append((my_x, my_y, (my_z + k) % N_Z))
                peers.append((1 - my_x, my_y, (my_z + k) % N_Z))
            return peers

        gathers = []
        if _VARIANT != "comm":
            base = my_y * V_PER
            for i in range(R):
                vid = ids_ref[row0 + i, 0] - base
                vid = jnp.clip(vid, 0, V_PER - 1)
                cp = pltpu.make_async_copy(
                    e_ref.at[pl.ds(vid, 1), :],
                    g.at[pl.ds(i, 1), :],
                    gsem.at[i % N_GSEM],
                )
                cp.start()
                gathers.append(cp)

        if _VARIANT != "compute":
            barrier_sem = pltpu.get_barrier_semaphore()
            for k in range(1, N_Y):
                pl.semaphore_signal(
                    barrier_sem, inc=1,
                    device_id=(my_x, (my_y + k) % N_Y, my_z),
                    device_id_type=pl.DeviceIdType.MESH,
                )
            for p in xz_peers():
                pl.semaphore_signal(
                    barrier_sem, inc=1,
                    device_id=p, device_id_type=pl.DeviceIdType.MESH,
                )

        if _VARIANT != "comm":
            for cp in gathers:
                cp.wait()
            lid = idv_ref[pl.ds(row0, R), :] - my_y * V_PER
            mask = ((lid >= 0) & (lid < V_PER)).astype(jnp.float32)
            yb[my_y] = (g[:, :] * mask).astype(jnp.bfloat16)
        else:
            yb[my_y] = jnp.zeros((R, D), jnp.bfloat16)
        if _VARIANT == "compute":
            out_ref[:, :] = jnp.zeros((T, D), jnp.float32)
            out_ref[pl.ds(row0, R), :] = yb[my_y].astype(jnp.float32)
            return

        pl.semaphore_wait(barrier_sem, 10)

        y_sends = []
        for k in range(1, N_Y):
            r = pltpu.make_async_remote_copy(
                src_ref=yb.at[my_y], dst_ref=yb.at[my_y],
                send_sem=ys.at[k - 1], recv_sem=yr.at[my_y],
                device_id=(my_x, (my_y + k) % N_Y, my_z),
                device_id_type=pl.DeviceIdType.MESH,
            )
            r.start()
            y_sends.append(r)
        for k in range(1, N_Y):
            src_y = (my_y + k) % N_Y
            pltpu.make_async_remote_copy(
                src_ref=yb.at[my_y], dst_ref=yb.at[src_y],
                send_sem=ys.at[k - 1], recv_sem=yr.at[src_y],
                device_id=(my_x, src_y, my_z),
                device_id_type=pl.DeviceIdType.MESH,
            ).wait_recv()
        fin = yb[0] + yb[1] + yb[2] + yb[3]
        fb[my_s] = fin

        f_sends = []
        for i, p in enumerate(xz_peers()):
            r = pltpu.make_async_remote_copy(
                src_ref=fb.at[my_s], dst_ref=fb.at[my_s],
                send_sem=fs.at[i], recv_sem=fr.at[my_s],
                device_id=p, device_id_type=pl.DeviceIdType.MESH,
            )
            r.start()
            f_sends.append(r)
        out_ref[pl.ds(row0, R), :] = fin.astype(jnp.float32)
        for i, p in enumerate(xz_peers()):
            src_s = p[0] * N_Z + p[2]
            pltpu.make_async_remote_copy(
                src_ref=fb.at[my_s], dst_ref=fb.at[src_s],
                send_sem=fs.at[i], recv_sem=fr.at[src_s],
                device_id=p, device_id_type=pl.DeviceIdType.MESH,
            ).wait_recv()
        for j in range(N_S):
            out_ref[pl.ds(j * R, R), :] = fb[j].astype(jnp.float32)

        for r in y_sends + f_sends:
            r.wait_send()

    return pl.pallas_call(
        body,
        out_shape=jax.ShapeDtypeStruct((T, D), jnp.float32),
        in_specs=[
            pl.BlockSpec(memory_space=pltpu.VMEM),
            pl.BlockSpec(memory_space=pltpu.MemorySpace.SMEM),
            pl.BlockSpec(memory_space=pl.ANY),
        ],
        out_specs=pl.BlockSpec(memory_space=pltpu.VMEM),
        scratch_shapes=[
            pltpu.VMEM((R, D), jnp.float32),
            pltpu.VMEM((N_Y, R, D), jnp.bfloat16),
            pltpu.VMEM((N_S, R, D), jnp.bfloat16),
            pltpu.SemaphoreType.DMA((N_GSEM,)),
            pltpu.SemaphoreType.DMA((N_Y - 1,)),
            pltpu.SemaphoreType.DMA((N_Y,)),
            pltpu.SemaphoreType.DMA((N_S - 1,)),
            pltpu.SemaphoreType.DMA((N_S,)),
        ],
        compiler_params=(
            pltpu.CompilerParams(collective_id=0)
            if _VARIANT != "compute"
            else pltpu.CompilerParams()
        ),
    )(ids2, ids2, E)
